# Initial kernel scaffold; baseline (speedup 1.0000x reference)
import jax
import jax.numpy as jnp
from jax.experimental import pallas as pl


def kernel(pd_scores, pd_bboxes, anc_points, gt_labels, gt_bboxes, mask_gt):
    raise NotImplementedError("write your pallas kernel here")


# speedup vs baseline: 18.8545x; 18.8545x over previous
import functools
import math

import jax
import jax.numpy as jnp
from jax.experimental import pallas as pl

NUM_CLASSES = 80
MAX_ANCHOR_MATCHES = 10
EPSILON = 1e-9
CIOU_EPS = 1e-7
_INV_PI2_4 = 4.0 / (math.pi ** 2)
_HIGHEST = jax.lax.Precision.HIGHEST


def _assigner_kernel(ps_ref, pbt_ref, anc_ref, lab_ref, gtb_ref, mgt_ref,
                     atp_ref, atg_ref, tb_ref, ts_ref, fg_ref):
    G = lab_ref.shape[1]
    A = anc_ref.shape[1]
    C = ps_ref.shape[2]

    gtb = gtb_ref[0]
    x1g, y1g = gtb[:, 0:1], gtb[:, 1:2]
    x2g, y2g = gtb[:, 2:3], gtb[:, 3:4]
    pbt = pbt_ref[0]
    x1p, y1p = pbt[0:1, :], pbt[1:2, :]
    x2p, y2p = pbt[2:3, :], pbt[3:4, :]
    ax = anc_ref[0:1, :]
    ay = anc_ref[1:2, :]
    mgt = mgt_ref[0]
    lab = lab_ref[0]

    mind = jnp.minimum(jnp.minimum(ax - x1g, ay - y1g),
                       jnp.minimum(x2g - ax, y2g - ay))
    mask = (mind > EPSILON) & (mgt > 0.5)

    iota_c = jax.lax.broadcasted_iota(jnp.int32, (G, C), 1)
    onehot_cls = (jnp.maximum(lab, 0) == iota_c).astype(jnp.float32)
    ps = jax.lax.dot_general(onehot_cls, ps_ref[0],
                             (((1,), (1,)), ((), ())),
                             precision=_HIGHEST,
                             preferred_element_type=jnp.float32)
    bbox_scores = jnp.where(mask, ps, 0.0)

    w1 = x2g - x1g
    h1 = y2g - y1g + CIOU_EPS
    w2 = x2p - x1p
    h2 = y2p - y1p + CIOU_EPS
    inter_w = jnp.maximum(jnp.minimum(x2g, x2p) - jnp.maximum(x1g, x1p), 0.0)
    inter_h = jnp.maximum(jnp.minimum(y2g, y2p) - jnp.maximum(y1g, y1p), 0.0)
    inter = inter_w * inter_h
    union = w1 * h1 + w2 * h2 - inter + CIOU_EPS
    iou = inter / union
    cw = jnp.maximum(x2g, x2p) - jnp.minimum(x1g, x1p)
    ch = jnp.maximum(y2g, y2p) - jnp.minimum(y1g, y1p)
    c2 = cw ** 2 + ch ** 2 + CIOU_EPS
    rho2 = ((x1p + x2p - x1g - x2g) ** 2 + (y1p + y2p - y1g - y2g) ** 2) / 4.0
    atan_p = atp_ref[0]
    atan_g = atg_ref[0]
    v = _INV_PI2_4 * (atan_p - atan_g) ** 2
    alpha = v / (v - iou + (1.0 + CIOU_EPS))
    ciou = iou - (rho2 / c2 + v * alpha)
    overlaps = jnp.where(mask, ciou, 0.0)

    o2 = overlaps * overlaps
    align = jnp.sqrt(bbox_scores) * (o2 * o2 * o2)

    iota_a = jax.lax.broadcasted_iota(jnp.int32, (G, A), 1)
    work = align
    mask_topk = jnp.zeros((G, A), jnp.float32)
    for _ in range(MAX_ANCHOR_MATCHES):
        rowmax = jnp.max(work, axis=1, keepdims=True)
        hit_idx = jnp.min(jnp.where(work == rowmax, iota_a, A),
                          axis=1, keepdims=True)
        hit = iota_a == hit_idx
        mask_topk = mask_topk + hit.astype(jnp.float32)
        work = jnp.where(hit, -1.0, work)

    mask_pos = mask_topk * mask.astype(jnp.float32)

    fg = jnp.sum(mask_pos, axis=0, keepdims=True)
    iota_g = jax.lax.broadcasted_iota(jnp.int32, (G, A), 0)
    omax = jnp.max(overlaps, axis=0, keepdims=True)
    gidx = jnp.min(jnp.where(overlaps == omax, iota_g, G),
                   axis=0, keepdims=True)
    onehot_best = (iota_g == gidx).astype(jnp.float32)
    mask_pos = jnp.where(fg > 1.0, onehot_best, mask_pos)
    fg = jnp.sum(mask_pos, axis=0, keepdims=True)
    fg_ref[0] = fg

    align2 = align * mask_pos
    pos_align = jnp.max(align2, axis=1, keepdims=True)
    pos_over = jnp.max(overlaps * mask_pos, axis=1, keepdims=True)
    norm = jnp.max(align2 * pos_over / (pos_align + EPSILON),
                   axis=0, keepdims=True)

    ts_ref[0] = jax.lax.dot_general(mask_pos * norm, onehot_cls,
                                    (((0,), (0,)), ((), ())),
                                    precision=_HIGHEST,
                                    preferred_element_type=jnp.float32)
    mask_pos_plus = mask_pos + (iota_g == 0).astype(jnp.float32) * (1.0 - fg)
    tb_ref[0] = jax.lax.dot_general(mask_pos_plus, gtb,
                                    (((0,), (0,)), ((), ())),
                                    precision=_HIGHEST,
                                    preferred_element_type=jnp.float32)


@jax.jit
def kernel(pd_scores, pd_bboxes, anc_points, gt_labels, gt_bboxes, mask_gt):
    B, A, C = pd_scores.shape
    G = gt_labels.shape[1]
    pbt = jnp.transpose(pd_bboxes, (0, 2, 1))
    anc_t = jnp.transpose(anc_points, (1, 0))
    lab3 = gt_labels.reshape(B, G, 1)
    mgt3 = mask_gt.astype(jnp.float32).reshape(B, G, 1)
    atan_p = jnp.arctan((pd_bboxes[..., 2] - pd_bboxes[..., 0]) /
                        (pd_bboxes[..., 3] - pd_bboxes[..., 1] + CIOU_EPS))
    atan_p = atan_p.reshape(B, 1, A)
    atan_g = jnp.arctan((gt_bboxes[..., 2] - gt_bboxes[..., 0]) /
                        (gt_bboxes[..., 3] - gt_bboxes[..., 1] + CIOU_EPS))
    atan_g = atan_g.reshape(B, G, 1)

    out_shapes = (
        jax.ShapeDtypeStruct((B, A, 4), jnp.float32),
        jax.ShapeDtypeStruct((B, A, C), jnp.float32),
        jax.ShapeDtypeStruct((B, 1, A), jnp.float32),
    )
    tb, ts, fg = pl.pallas_call(
        _assigner_kernel,
        grid=(B,),
        in_specs=[
            pl.BlockSpec((1, A, C), lambda b: (b, 0, 0)),
            pl.BlockSpec((1, 4, A), lambda b: (b, 0, 0)),
            pl.BlockSpec((2, A), lambda b: (0, 0)),
            pl.BlockSpec((1, G, 1), lambda b: (b, 0, 0)),
            pl.BlockSpec((1, G, 4), lambda b: (b, 0, 0)),
            pl.BlockSpec((1, G, 1), lambda b: (b, 0, 0)),
            pl.BlockSpec((1, 1, A), lambda b: (b, 0, 0)),
            pl.BlockSpec((1, G, 1), lambda b: (b, 0, 0)),
        ],
        out_specs=[
            pl.BlockSpec((1, A, 4), lambda b: (b, 0, 0)),
            pl.BlockSpec((1, A, C), lambda b: (b, 0, 0)),
            pl.BlockSpec((1, 1, A), lambda b: (b, 0, 0)),
        ],
        out_shape=out_shapes,
    )(pd_scores, pbt, anc_t, lab3, gt_bboxes, mgt3, atan_p, atan_g)
    return tb, ts, fg.reshape(B, A)
